# baseline (speedup 1.0000x reference)
import functools

import jax
import jax.numpy as jnp
from jax import lax
from jax.experimental import pallas as pl
from jax.experimental.pallas import tpu as pltpu
from jax.experimental.pallas import tpu_sc as plsc

N_ROWS = 4096
N_COLS = 26
D = 128
NC = 2
NS = 16
NW = NC * NS
RB = N_ROWS // NW
NBUF = 6

_mesh = plsc.VectorSubcoreMesh(core_axis_name="c", subcore_axis_name="s")


@functools.partial(
    pl.kernel,
    mesh=_mesh,
    out_type=jax.ShapeDtypeStruct((N_COLS, N_ROWS, D), jnp.float32),
    scratch_types=[
        pltpu.VMEM((N_COLS, RB), jnp.int32),
    ] + [pltpu.VMEM((RB, D), jnp.float32) for _ in range(NBUF)]
      + [pltpu.SemaphoreType.DMA for _ in range(2 * NBUF)],
)
def _gather_kernel(idx_hbm, table_hbm, out_hbm, idx_v, *bufs_sems):
    bufs = bufs_sems[:NBUF]
    g_sems = bufs_sems[NBUF:2 * NBUF]
    s_sems = bufs_sems[2 * NBUF:]
    wid = lax.axis_index("s") * NC + lax.axis_index("c")
    rbase = wid * RB
    pltpu.sync_copy(idx_hbm.at[:, pl.ds(rbase, RB)], idx_v)

    def gather(c, b):
        pltpu.async_copy(table_hbm.at[idx_v.at[c]], bufs[b], g_sems[b])

    def store(c, b):
        pltpu.async_copy(bufs[b], out_hbm.at[c, pl.ds(rbase, RB)], s_sems[b])

    def wait_gather(b):
        pltpu.make_async_copy(table_hbm.at[pl.ds(0, RB)], bufs[b],
                              g_sems[b]).wait()

    def wait_store(b):
        pltpu.make_async_copy(bufs[b], out_hbm.at[0, pl.ds(rbase, RB)],
                              s_sems[b]).wait()

    for b in range(NBUF):
        gather(b, b)

    def outer(g, carry):
        c0 = g * NBUF
        for b in range(NBUF):
            wait_gather(b)
            store(c0 + b, b)
        for b in range(NBUF):
            wait_store(b)
            gather(c0 + NBUF + b, b)
        return carry

    lax.fori_loop(0, N_COLS // NBUF - 1, outer, 0)

    c0 = (N_COLS // NBUF - 1) * NBUF
    for b in range(NBUF):
        wait_gather(b)
        store(c0 + b, b)
    for b in range(N_COLS - c0 - NBUF):
        wait_store(b)
        gather(c0 + NBUF + b, b)
    for b in range(2):
        wait_gather(b)
        store(c0 + NBUF + b, b)
    for b in range(NBUF):
        wait_store(b)


def kernel(x, table):
    out = _gather_kernel(x.T.astype(jnp.int32), table)
    return out.transpose(1, 0, 2)
